# baseline (speedup 1.0000x reference)
import jax
import jax.numpy as jnp
from jax import lax
from jax.experimental import pallas as pl
from jax.experimental.pallas import tpu as pltpu
from jax.experimental.pallas import tpu_sc as plsc

B = 16384
D = 32
N_NUM = 13
N_CAT = 26
N_TOK = N_NUM + 1 + N_CAT
PER_FIELD = 100000
TOTAL_CAT = N_CAT * PER_FIELD

NC = 2
NS = 16
NW = NC * NS
BB = 128
BLOCKS_PER_W = B // NW // BB


def _body(xt_hbm, nw_hbm, emb_hbm, bias_hbm, out_hbm,
          xt_v, nw_v, bias_v, idx2, stage2, biasx, numb,
          gsem0, gsem1, osem0, osem1):
    wid = lax.axis_index("s") * NC + lax.axis_index("c")
    pltpu.sync_copy(nw_hbm, nw_v)
    pltpu.sync_copy(bias_hbm, bias_v)

    def load_row128(ref, r):
        return [ref[r, pl.ds(l * 16, 16)] for l in range(8)]

    def gen_and_fire(f, buf, gsem):
        xc = load_row128(xt_v, N_NUM + f)
        idv = [xc[l].astype(jnp.int32) + f * PER_FIELD for l in range(8)]

        def fill_d(d, carry):
            for l in range(8):
                idx2[buf, d, pl.ds(l * 16, 16)] = idv[l] + d * TOTAL_CAT
            return carry
        lax.fori_loop(0, D, fill_d, None, unroll=4)
        for d in range(D):
            pltpu.async_copy(emb_hbm.at[idx2.at[buf, d]],
                             stage2.at[buf, d], gsem)

    def fill_biasx(f):
        br = [bias_v[N_NUM + f, pl.ds(0, 16)], bias_v[N_NUM + f, pl.ds(16, 16)]]
        for d in range(D):
            biasx[d, pl.ds(0, 16)] = jnp.full((16,), br[d // 16][d % 16],
                                              jnp.float32)

    def finish(f, buf, base_b, gsem, osem):
        pltpu.make_async_copy(out_hbm.at[pl.ds(0, D), pl.ds(0, BB)],
                              stage2.at[buf], gsem).wait()
        fill_biasx(f)

        def add_d(d, carry):
            bv = biasx[d, pl.ds(0, 16)]
            for l in range(8):
                sl = pl.ds(l * 16, 16)
                stage2[buf, d, sl] = stage2[buf, d, sl] + bv
            return carry
        lax.fori_loop(0, D, add_d, None, unroll=4)
        pltpu.async_copy(stage2.at[buf],
                         out_hbm.at[pl.ds((N_NUM + 1 + f) * D, D),
                                    pl.ds(base_b, BB)], osem)

    def wait_out(osem):
        pltpu.make_async_copy(stage2.at[0],
                              out_hbm.at[pl.ds(0, D), pl.ds(0, BB)],
                              osem).wait()

    def bblock(bb, _):
        base_b = wid * (BB * BLOCKS_PER_W) + bb * BB
        pltpu.sync_copy(xt_hbm.at[:, pl.ds(base_b, BB)], xt_v)

        gen_and_fire(0, 0, gsem0)

        nr = [nw_v[0, pl.ds(0, 16)], nw_v[0, pl.ds(16, 16)]]
        for d in range(D):
            s = nr[d // 16][d % 16]
            v = jnp.full((16,), s, jnp.float32)
            for l in range(8):
                numb[d, pl.ds(l * 16, 16)] = v

        def num_j(j, carry):
            xv = load_row128(xt_v, j - 1)
            nrj = [nw_v[j, pl.ds(0, 16)], nw_v[j, pl.ds(16, 16)]]
            brj = [bias_v[j - 1, pl.ds(0, 16)], bias_v[j - 1, pl.ds(16, 16)]]
            for d in range(D):
                s = nrj[d // 16][d % 16]
                bs = brj[d // 16][d % 16]
                for l in range(8):
                    numb[j * D + d, pl.ds(l * 16, 16)] = xv[l] * s + bs
            return carry
        lax.fori_loop(1, N_NUM + 1, num_j, None)
        pltpu.sync_copy(numb, out_hbm.at[pl.ds(0, (N_NUM + 1) * D),
                                         pl.ds(base_b, BB)])

        def pair(m, carry):
            f = 2 * m

            @pl.when(m > 0)
            def _():
                wait_out(osem1)
            gen_and_fire(f + 1, 1, gsem1)
            finish(f, 0, base_b, gsem0, osem0)

            @pl.when(m < 12)
            def _():
                wait_out(osem0)
                gen_and_fire(f + 2, 0, gsem0)
            finish(f + 1, 1, base_b, gsem1, osem1)
            return carry
        lax.fori_loop(0, 13, pair, None)
        wait_out(osem0)
        wait_out(osem1)
        return _

    lax.fori_loop(0, BLOCKS_PER_W, bblock, None)


@jax.jit
def kernel(x, num_weight, emb_table, bias):
    xt = x.T
    emb_flat = emb_table.T.reshape(-1)
    run = pl.kernel(
        _body,
        out_type=jax.ShapeDtypeStruct((N_TOK * D, B), jnp.float32),
        mesh=plsc.VectorSubcoreMesh(core_axis_name="c", subcore_axis_name="s"),
        compiler_params=pltpu.CompilerParams(use_tc_tiling_on_sc=False),
        scratch_types=[
            pltpu.VMEM((N_NUM + N_CAT, BB), jnp.float32),
            pltpu.VMEM((N_NUM + 1, D), jnp.float32),
            pltpu.VMEM((N_NUM + N_CAT, D), jnp.float32),
            pltpu.VMEM((2, D, BB), jnp.int32),
            pltpu.VMEM((2, D, BB), jnp.float32),
            pltpu.VMEM((D, 16), jnp.float32),
            pltpu.VMEM(((N_NUM + 1) * D, BB), jnp.float32),
            pltpu.SemaphoreType.DMA,
            pltpu.SemaphoreType.DMA,
            pltpu.SemaphoreType.DMA,
            pltpu.SemaphoreType.DMA,
        ],
    )
    out = run(xt, num_weight, emb_flat, bias)
    return out.T.reshape(B, N_TOK, D)


# speedup vs baseline: 1.0003x; 1.0003x over previous
import jax
import jax.numpy as jnp
from jax import lax
from jax.experimental import pallas as pl
from jax.experimental.pallas import tpu as pltpu
from jax.experimental.pallas import tpu_sc as plsc

B = 16384
D = 32
N_NUM = 13
N_CAT = 26
N_TOK = N_NUM + 1 + N_CAT
PER_FIELD = 100000
TOTAL_CAT = N_CAT * PER_FIELD

NC = 2
NS = 16
NW = NC * NS
BB = 128
BLOCKS_PER_W = B // NW // BB


def _body(xt_hbm, nw_hbm, emb_hbm, bias_hbm, out_hbm,
          xt_v, nw_v, bias_v, idx2, stage2, biasx, numb,
          gsem0, gsem1, osem0, osem1):
    wid = lax.axis_index("s") * NC + lax.axis_index("c")
    pltpu.sync_copy(nw_hbm, nw_v)
    pltpu.sync_copy(bias_hbm, bias_v)

    def load_row128(ref, r):
        return [ref[r, pl.ds(l * 16, 16)] for l in range(8)]

    def gen_and_fire(f, buf, gsem):
        xc = load_row128(xt_v, N_NUM + f)
        for l in range(8):
            idx2[buf, pl.ds(l * 16, 16)] = xc[l].astype(jnp.int32) + f * PER_FIELD
        for d in range(D):
            pltpu.async_copy(emb_hbm.at[d].at[idx2.at[buf]],
                             stage2.at[buf, d], gsem)

    def fill_biasx(f):
        br = [bias_v[N_NUM + f, pl.ds(0, 16)], bias_v[N_NUM + f, pl.ds(16, 16)]]
        for d in range(D):
            biasx[d, pl.ds(0, 16)] = jnp.full((16,), br[d // 16][d % 16],
                                              jnp.float32)

    def finish(f, buf, base_b, gsem, osem):
        pltpu.make_async_copy(out_hbm.at[pl.ds(0, D), pl.ds(0, BB)],
                              stage2.at[buf], gsem).wait()
        fill_biasx(f)

        def add_d(d, carry):
            bv = biasx[d, pl.ds(0, 16)]
            for l in range(8):
                sl = pl.ds(l * 16, 16)
                stage2[buf, d, sl] = stage2[buf, d, sl] + bv
            return carry
        lax.fori_loop(0, D, add_d, None, unroll=4)
        pltpu.async_copy(stage2.at[buf],
                         out_hbm.at[pl.ds((N_NUM + 1 + f) * D, D),
                                    pl.ds(base_b, BB)], osem)

    def wait_out(osem):
        pltpu.make_async_copy(stage2.at[0],
                              out_hbm.at[pl.ds(0, D), pl.ds(0, BB)],
                              osem).wait()

    def bblock(bb, _):
        base_b = wid * (BB * BLOCKS_PER_W) + bb * BB
        pltpu.sync_copy(xt_hbm.at[:, pl.ds(base_b, BB)], xt_v)

        gen_and_fire(0, 0, gsem0)

        nr = [nw_v[0, pl.ds(0, 16)], nw_v[0, pl.ds(16, 16)]]
        for d in range(D):
            s = nr[d // 16][d % 16]
            v = jnp.full((16,), s, jnp.float32)
            for l in range(8):
                numb[d, pl.ds(l * 16, 16)] = v

        def num_j(j, carry):
            xv = load_row128(xt_v, j - 1)
            nrj = [nw_v[j, pl.ds(0, 16)], nw_v[j, pl.ds(16, 16)]]
            brj = [bias_v[j - 1, pl.ds(0, 16)], bias_v[j - 1, pl.ds(16, 16)]]
            for d in range(D):
                s = nrj[d // 16][d % 16]
                bs = brj[d // 16][d % 16]
                for l in range(8):
                    numb[j * D + d, pl.ds(l * 16, 16)] = xv[l] * s + bs
            return carry
        lax.fori_loop(1, N_NUM + 1, num_j, None)
        pltpu.sync_copy(numb, out_hbm.at[pl.ds(0, (N_NUM + 1) * D),
                                         pl.ds(base_b, BB)])

        def pair(m, carry):
            f = 2 * m

            @pl.when(m > 0)
            def _():
                wait_out(osem1)
            gen_and_fire(f + 1, 1, gsem1)
            finish(f, 0, base_b, gsem0, osem0)

            @pl.when(m < 12)
            def _():
                wait_out(osem0)
                gen_and_fire(f + 2, 0, gsem0)
            finish(f + 1, 1, base_b, gsem1, osem1)
            return carry
        lax.fori_loop(0, 13, pair, None)
        wait_out(osem0)
        wait_out(osem1)
        return _

    lax.fori_loop(0, BLOCKS_PER_W, bblock, None)


@jax.jit
def kernel(x, num_weight, emb_table, bias):
    xt = x.T
    emb_t = emb_table.T
    run = pl.kernel(
        _body,
        out_type=jax.ShapeDtypeStruct((N_TOK * D, B), jnp.float32),
        mesh=plsc.VectorSubcoreMesh(core_axis_name="c", subcore_axis_name="s"),
        compiler_params=pltpu.CompilerParams(use_tc_tiling_on_sc=False),
        scratch_types=[
            pltpu.VMEM((N_NUM + N_CAT, BB), jnp.float32),
            pltpu.VMEM((N_NUM + 1, D), jnp.float32),
            pltpu.VMEM((N_NUM + N_CAT, D), jnp.float32),
            pltpu.VMEM((2, BB), jnp.int32),
            pltpu.VMEM((2, D, BB), jnp.float32),
            pltpu.VMEM((D, 16), jnp.float32),
            pltpu.VMEM(((N_NUM + 1) * D, BB), jnp.float32),
            pltpu.SemaphoreType.DMA,
            pltpu.SemaphoreType.DMA,
            pltpu.SemaphoreType.DMA,
            pltpu.SemaphoreType.DMA,
        ],
    )
    out = run(xt, num_weight, emb_t, bias)
    return out.T.reshape(B, N_TOK, D)


# speedup vs baseline: 7.9281x; 7.9257x over previous
import jax
import jax.numpy as jnp
from jax import lax
from jax.experimental import pallas as pl
from jax.experimental.pallas import tpu as pltpu
from jax.experimental.pallas import tpu_sc as plsc

B = 16384
D = 32
N_NUM = 13
N_CAT = 26
N_TOK = N_NUM + 1 + N_CAT
PER_FIELD = 100000
TOTAL_CAT = N_CAT * PER_FIELD

NC = 2
NS = 16
NW = NC * NS
BB = 128
BLOCKS_PER_W = B // NW // BB

KB = 8192
NFULL = TOTAL_CAT // KB
KTAIL = TOTAL_CAT - NFULL * KB
TAIL_W = NFULL % NW


def _detile(emb_hbm, out_hbm, vb_a, vb_b, vtail, isem0, isem1, osem0, osem1):
    vbufs = (vb_a, vb_b)
    wid = lax.axis_index("s") * NC + lax.axis_index("c")
    nblocks = jnp.where(wid < NFULL - (NFULL // NW) * NW, NFULL // NW + 1,
                        NFULL // NW)
    nunits = nblocks * D

    def fire_in(u, buf, isem):
        blk = wid + (u // D) * NW
        d = u % D
        k0 = blk * KB
        pltpu.async_copy(emb_hbm.at[d, pl.ds(k0, KB)], vbufs[buf], isem)

    def fire_out(u, buf, isem, osem):
        blk = wid + (u // D) * NW
        d = u % D
        k0 = blk * KB
        pltpu.make_async_copy(emb_hbm.at[0, pl.ds(0, KB)], vbufs[buf],
                              isem).wait()
        pltpu.async_copy(vbufs[buf],
                         out_hbm.at[pl.ds(d * TOTAL_CAT + k0, KB)], osem)

    def wait_out(osem):
        pltpu.make_async_copy(vb_a, out_hbm.at[pl.ds(0, KB)], osem).wait()

    @pl.when(nunits > 0)
    def _():
        fire_in(0, 0, isem0)

        npair = nunits // 2

        def pair(m, carry):
            u0 = 2 * m

            @pl.when(m > 0)
            def _():
                wait_out(osem1)
            fire_in(u0 + 1, 1, isem1)
            fire_out(u0, 0, isem0, osem0)

            @pl.when(m < npair - 1)
            def _():
                wait_out(osem0)
                fire_in(u0 + 2, 0, isem0)
            fire_out(u0 + 1, 1, isem1, osem1)
            return carry
        lax.fori_loop(0, npair, pair, None)
        wait_out(osem0)
        wait_out(osem1)

    @pl.when(wid == TAIL_W)
    def _():
        k0 = NFULL * KB

        def tail_d(d, carry):
            pltpu.sync_copy(emb_hbm.at[d, pl.ds(k0, KTAIL)], vtail)
            pltpu.sync_copy(vtail,
                            out_hbm.at[pl.ds(d * TOTAL_CAT + k0, KTAIL)])
            return carry
        lax.fori_loop(0, D, tail_d, None)


def _gather_body(xt_hbm, nw_hbm, emb_hbm, bias_hbm, out_hbm,
                 xt_v, nw_v, bias_v, idx2, stage2, biasx, numb,
                 gsem0, gsem1, osem0, osem1):
    wid = lax.axis_index("s") * NC + lax.axis_index("c")
    pltpu.sync_copy(nw_hbm, nw_v)
    pltpu.sync_copy(bias_hbm, bias_v)

    def load_row128(ref, r):
        return [ref[r, pl.ds(l * 16, 16)] for l in range(8)]

    def gen_and_fire(f, buf, gsem):
        xc = load_row128(xt_v, N_NUM + f)
        for l in range(8):
            idx2[buf, pl.ds(l * 16, 16)] = xc[l].astype(jnp.int32) + f * PER_FIELD
        for d in range(D):
            pltpu.async_copy(emb_hbm.at[d].at[idx2.at[buf]],
                             stage2.at[buf, d], gsem)

    def fill_biasx(f):
        br = [bias_v[N_NUM + f, pl.ds(0, 16)], bias_v[N_NUM + f, pl.ds(16, 16)]]
        for d in range(D):
            biasx[d, pl.ds(0, 16)] = jnp.full((16,), br[d // 16][d % 16],
                                              jnp.float32)

    def finish(f, buf, base_b, gsem, osem):
        pltpu.make_async_copy(out_hbm.at[pl.ds(0, D), pl.ds(0, BB)],
                              stage2.at[buf], gsem).wait()
        fill_biasx(f)

        def add_d(d, carry):
            bv = biasx[d, pl.ds(0, 16)]
            for l in range(8):
                sl = pl.ds(l * 16, 16)
                stage2[buf, d, sl] = stage2[buf, d, sl] + bv
            return carry
        lax.fori_loop(0, D, add_d, None, unroll=4)
        pltpu.async_copy(stage2.at[buf],
                         out_hbm.at[pl.ds((N_NUM + 1 + f) * D, D),
                                    pl.ds(base_b, BB)], osem)

    def wait_out(osem):
        pltpu.make_async_copy(stage2.at[0],
                              out_hbm.at[pl.ds(0, D), pl.ds(0, BB)],
                              osem).wait()

    def bblock(bb, _):
        base_b = wid * (BB * BLOCKS_PER_W) + bb * BB
        pltpu.sync_copy(xt_hbm.at[:, pl.ds(base_b, BB)], xt_v)

        gen_and_fire(0, 0, gsem0)

        nr = [nw_v[0, pl.ds(0, 16)], nw_v[0, pl.ds(16, 16)]]
        for d in range(D):
            s = nr[d // 16][d % 16]
            v = jnp.full((16,), s, jnp.float32)
            for l in range(8):
                numb[d, pl.ds(l * 16, 16)] = v

        def num_j(j, carry):
            xv = load_row128(xt_v, j - 1)
            nrj = [nw_v[j, pl.ds(0, 16)], nw_v[j, pl.ds(16, 16)]]
            brj = [bias_v[j - 1, pl.ds(0, 16)], bias_v[j - 1, pl.ds(16, 16)]]
            for d in range(D):
                s = nrj[d // 16][d % 16]
                bs = brj[d // 16][d % 16]
                for l in range(8):
                    numb[j * D + d, pl.ds(l * 16, 16)] = xv[l] * s + bs
            return carry
        lax.fori_loop(1, N_NUM + 1, num_j, None)
        pltpu.sync_copy(numb, out_hbm.at[pl.ds(0, (N_NUM + 1) * D),
                                         pl.ds(base_b, BB)])

        def pair(m, carry):
            f = 2 * m

            @pl.when(m > 0)
            def _():
                wait_out(osem1)
            gen_and_fire(f + 1, 1, gsem1)
            finish(f, 0, base_b, gsem0, osem0)

            @pl.when(m < 12)
            def _():
                wait_out(osem0)
                gen_and_fire(f + 2, 0, gsem0)
            finish(f + 1, 1, base_b, gsem1, osem1)
            return carry
        lax.fori_loop(0, 13, pair, None)
        wait_out(osem0)
        wait_out(osem1)
        return _

    lax.fori_loop(0, BLOCKS_PER_W, bblock, None)


@jax.jit
def kernel(x, num_weight, emb_table, bias):
    xt = x.T
    emb_t = emb_table.T

    detile = pl.kernel(
        _detile,
        out_type=jax.ShapeDtypeStruct((D * TOTAL_CAT,), jnp.float32),
        mesh=plsc.VectorSubcoreMesh(core_axis_name="c", subcore_axis_name="s"),
        compiler_params=pltpu.CompilerParams(use_tc_tiling_on_sc=True),
        scratch_types=[
            pltpu.VMEM((KB,), jnp.float32),
            pltpu.VMEM((KB,), jnp.float32),
            pltpu.VMEM((KTAIL,), jnp.float32),
            pltpu.SemaphoreType.DMA,
            pltpu.SemaphoreType.DMA,
            pltpu.SemaphoreType.DMA,
            pltpu.SemaphoreType.DMA,
        ],
    )
    emb_lin = detile(emb_t).reshape(D, TOTAL_CAT)

    run = pl.kernel(
        _gather_body,
        out_type=jax.ShapeDtypeStruct((N_TOK * D, B), jnp.float32),
        mesh=plsc.VectorSubcoreMesh(core_axis_name="c", subcore_axis_name="s"),
        compiler_params=pltpu.CompilerParams(use_tc_tiling_on_sc=False),
        scratch_types=[
            pltpu.VMEM((N_NUM + N_CAT, BB), jnp.float32),
            pltpu.VMEM((N_NUM + 1, D), jnp.float32),
            pltpu.VMEM((N_NUM + N_CAT, D), jnp.float32),
            pltpu.VMEM((2, BB), jnp.int32),
            pltpu.VMEM((2, D, BB), jnp.float32),
            pltpu.VMEM((D, 16), jnp.float32),
            pltpu.VMEM(((N_NUM + 1) * D, BB), jnp.float32),
            pltpu.SemaphoreType.DMA,
            pltpu.SemaphoreType.DMA,
            pltpu.SemaphoreType.DMA,
            pltpu.SemaphoreType.DMA,
        ],
    )
    out = run(xt, num_weight, emb_lin, bias)
    return out.T.reshape(B, N_TOK, D)
